# Initial kernel scaffold; baseline (speedup 1.0000x reference)
import jax
import jax.numpy as jnp
from jax.experimental import pallas as pl


def kernel(x, edge_index, edge_weight, W1, b1, W2, b2, W3, b3):
    raise NotImplementedError("write your pallas kernel here")


# speedup vs baseline: 15.3500x; 15.3500x over previous
import functools

import jax
import jax.numpy as jnp
from jax import lax
from jax.experimental import pallas as pl
from jax.experimental.pallas import tpu as pltpu
from jax.experimental.pallas import tpu_sc as plsc

NC = 2
NS = 16
NW = NC * NS
LANES = 16
CE = 128


def _mesh():
  return plsc.VectorSubcoreMesh(
      core_axis_name="c", subcore_axis_name="s", num_cores=NC,
      num_subcores=NS)


def _make_sc_deg(CH, Npad):
  sl_n = Npad // NS

  @functools.partial(
      pl.kernel, mesh=_mesh(),
      out_type=jax.ShapeDtypeStruct((NC, Npad), jnp.float32),
      scratch_types=[
          pltpu.VMEM((CH, CE), jnp.int32),
          pltpu.VMEM((CH, CE), jnp.float32),
          pltpu.VMEM((sl_n,), jnp.float32),
          pltpu.VMEM_SHARED((Npad,), jnp.float32),
      ],
      name="sc_deg",
  )
  def sc_deg(d_hbm, w_hbm, out_hbm, didx, wv, zv, deg_s):
    c = lax.axis_index("c")
    s = lax.axis_index("s")
    wid = s * NC + c
    z16 = jnp.zeros((LANES,), jnp.float32)

    def zb(i, carry):
      zv[pl.ds(i * LANES, LANES)] = z16
      return carry
    lax.fori_loop(0, sl_n // LANES, zb, 0)
    pltpu.sync_copy(zv, deg_s.at[pl.ds(s * sl_n, sl_n)])
    pltpu.sync_copy(d_hbm.at[wid], didx)
    pltpu.sync_copy(w_hbm.at[wid], wv)
    plsc.subcore_barrier()

    def ch(j, carry):
      pltpu.sync_copy(wv.at[j], deg_s.at[didx.at[j]], add=True)
      return carry
    lax.fori_loop(0, CH, ch, 0)
    plsc.subcore_barrier()
    pltpu.sync_copy(deg_s.at[pl.ds(s * sl_n, sl_n)],
                    out_hbm.at[c].at[pl.ds(s * sl_n, sl_n)])

  return sc_deg


def _make_sc_norm(CH):
  @functools.partial(
      pl.kernel, mesh=_mesh(),
      out_type=jax.ShapeDtypeStruct((NW, CH, CE), jnp.float32),
      scratch_types=[
          pltpu.VMEM((CH, CE), jnp.int32),
          pltpu.VMEM((CH, CE), jnp.int32),
          pltpu.VMEM((CH, CE), jnp.float32),
          pltpu.VMEM((CH, CE), jnp.float32),
          pltpu.VMEM((2, CE), jnp.float32),
          pltpu.VMEM((2, CE), jnp.float32),
          pltpu.SemaphoreType.DMA,
          pltpu.SemaphoreType.DMA,
      ],
      name="sc_norm",
  )
  def sc_norm(dinv_hbm, s_hbm, d_hbm, w_hbm, norm_hbm,
              sidx, didx, wv, nv, av, bv, sem0, sem1):
    c = lax.axis_index("c")
    s = lax.axis_index("s")
    wid = s * NC + c
    pltpu.sync_copy(s_hbm.at[wid], sidx)
    pltpu.sync_copy(d_hbm.at[wid], didx)
    pltpu.sync_copy(w_hbm.at[wid], wv)

    def fire(j, buf, sem):
      pltpu.async_copy(dinv_hbm.at[sidx.at[j]], av.at[buf], sem)
      pltpu.async_copy(dinv_hbm.at[didx.at[j]], bv.at[buf], sem)

    def process(j, buf, sem):
      pltpu.make_async_copy(dinv_hbm.at[sidx.at[j]], av.at[buf], sem).wait()
      pltpu.make_async_copy(dinv_hbm.at[didx.at[j]], bv.at[buf], sem).wait()

      def inner(k, carry2):
        sl = pl.ds(k * LANES, LANES)
        nv[j, sl] = av[buf, sl] * wv[j, sl] * bv[buf, sl]
        return carry2
      lax.fori_loop(0, CE // LANES, inner, 0)

      @pl.when(j + 2 < CH)
      def _():
        fire(j + 2, buf, sem)

    fire(0, 0, sem0)
    fire(1, 1, sem1)

    def step(m, carry):
      process(2 * m, 0, sem0)
      process(2 * m + 1, 1, sem1)
      return carry
    lax.fori_loop(0, CH // 2, step, 0)
    if CH % 2:
      process(jnp.int32(CH - 1), 0, sem0)
    pltpu.sync_copy(nv, norm_hbm.at[wid])

  return sc_norm


def _make_sc_agg(CH, Npad, Wd):
  sl_n = Npad // NS
  KW = Wd // LANES
  assert sl_n % CE == 0

  @functools.partial(
      pl.kernel, mesh=_mesh(),
      out_type=jax.ShapeDtypeStruct((NC, Npad, Wd), jnp.float32),
      scratch_types=[
          pltpu.VMEM((4, CE), jnp.int32),
          pltpu.VMEM((4, CE), jnp.int32),
          pltpu.VMEM((4, CE), jnp.float32),
          pltpu.VMEM((CE, Wd), jnp.float32),
          pltpu.VMEM((CE, Wd), jnp.float32),
          pltpu.SemaphoreType.DMA,
          pltpu.SemaphoreType.DMA,
          pltpu.SemaphoreType.DMA,
          pltpu.VMEM_SHARED((Npad, Wd), jnp.float32),
      ],
      name="sc_agg",
  )
  def sc_agg(hw_hbm, s_hbm, d_hbm, n_hbm, out_hbm,
             sidx, didx, nv, rows0, rows1, semi, semg0, semg1, acc_s):
    c = lax.axis_index("c")
    s = lax.axis_index("s")
    wid = s * NC + c
    z16 = jnp.zeros((LANES,), jnp.float32)

    def zr(i, carry):
      for k in range(KW):
        rows0[i, pl.ds(k * LANES, LANES)] = z16
      return carry
    lax.fori_loop(0, CE, zr, 0)
    for q in range(sl_n // CE):
      pltpu.sync_copy(rows0, acc_s.at[pl.ds(s * sl_n + q * CE, CE)])
    plsc.subcore_barrier()

    def fire_idx(j):
      slot = j & 3
      pltpu.async_copy(s_hbm.at[wid].at[j], sidx.at[slot], semi)
      pltpu.async_copy(d_hbm.at[wid].at[j], didx.at[slot], semi)
      pltpu.async_copy(n_hbm.at[wid].at[j], nv.at[slot], semi)

    def wait_idx(j):
      slot = j & 3
      pltpu.make_async_copy(s_hbm.at[wid].at[j], sidx.at[slot], semi).wait()
      pltpu.make_async_copy(d_hbm.at[wid].at[j], didx.at[slot], semi).wait()
      pltpu.make_async_copy(n_hbm.at[wid].at[j], nv.at[slot], semi).wait()

    def fire_gather(j, rows, semg):
      pltpu.async_copy(hw_hbm.at[sidx.at[j & 3]], rows, semg)

    for j in range(4):
      fire_idx(jnp.int32(j))
    wait_idx(jnp.int32(0))
    fire_gather(jnp.int32(0), rows0, semg0)
    wait_idx(jnp.int32(1))
    fire_gather(jnp.int32(1), rows1, semg1)

    def process(j, rows, semg):
      slot = j & 3
      pltpu.make_async_copy(hw_hbm.at[sidx.at[slot]], rows, semg).wait()

      def se(g, carry):
        n16 = nv[slot, pl.ds(g * LANES, LANES)]
        for l in range(LANES):
          e = g * LANES + l
          nrm = n16[l]
          for k in range(KW):
            sl = pl.ds(k * LANES, LANES)
            rows[e, sl] = rows[e, sl] * nrm
        return carry
      lax.fori_loop(0, CE // LANES, se, 0)
      pltpu.sync_copy(rows, acc_s.at[didx.at[slot]], add=True)

      @pl.when(j + 4 < CH)
      def _():
        fire_idx(j + 4)

      @pl.when(j + 2 < CH)
      def _():
        wait_idx(j + 2)
        fire_gather(j + 2, rows, semg)

    def step(m, carry):
      process(2 * m, rows0, semg0)
      process(2 * m + 1, rows1, semg1)
      return carry
    lax.fori_loop(0, CH // 2, step, 0)
    if CH % 2:
      process(jnp.int32(CH - 1), rows0, semg0)

    plsc.subcore_barrier()
    base = s * sl_n
    pltpu.sync_copy(acc_s.at[pl.ds(base, sl_n)],
                    out_hbm.at[c].at[pl.ds(base, sl_n)])

  return sc_agg


def _tc1(deg2, xp, W1, Npad, G):
  R = Npad // G
  Rd = R // 128
  D = xp.shape[1]
  H = W1.shape[1]

  def body(deg_ref, x_ref, w_ref, dinv_ref, hw_ref):
    deg = deg_ref[0] + deg_ref[1]
    dinv_ref[...] = jnp.where(
        deg > 0, lax.rsqrt(jnp.maximum(deg, 1e-12)), 0.0)
    hw_ref[...] = jnp.dot(x_ref[...], w_ref[...],
                          preferred_element_type=jnp.float32)

  return pl.pallas_call(
      body,
      grid=(G,),
      in_specs=[
          pl.BlockSpec((2, Rd, 128), lambda i: (0, i, 0)),
          pl.BlockSpec((R, D), lambda i: (i, 0)),
          pl.BlockSpec((D, H), lambda i: (0, 0)),
      ],
      out_specs=[
          pl.BlockSpec((Rd, 128), lambda i: (i, 0)),
          pl.BlockSpec((R, H), lambda i: (i, 0)),
      ],
      out_shape=[
          jax.ShapeDtypeStruct((Npad // 128, 128), jnp.float32),
          jax.ShapeDtypeStruct((Npad, H), jnp.float32),
      ],
  )(deg2, xp, W1)


def _tc_layer(acc, b, W, Npad, G):
  R = Npad // G
  H = acc.shape[2]
  Hn = W.shape[1]

  def body(acc_ref, b_ref, w_ref, out_ref):
    h = jnp.maximum(acc_ref[0] + acc_ref[1] + b_ref[...], 0.0)
    out_ref[...] = jnp.dot(h, w_ref[...],
                           preferred_element_type=jnp.float32)

  return pl.pallas_call(
      body,
      grid=(G,),
      in_specs=[
          pl.BlockSpec((2, R, H), lambda i: (0, i, 0)),
          pl.BlockSpec((1, H), lambda i: (0, 0)),
          pl.BlockSpec((H, Hn), lambda i: (0, 0)),
      ],
      out_specs=pl.BlockSpec((R, Hn), lambda i: (i, 0)),
      out_shape=jax.ShapeDtypeStruct((Npad, Hn), jnp.float32),
  )(acc, b, W)


def _tc_relu(acc, b, Npad, G):
  R = Npad // G
  H = acc.shape[2]

  def body(acc_ref, b_ref, out_ref):
    out_ref[...] = jnp.maximum(acc_ref[0] + acc_ref[1] + b_ref[...], 0.0)

  return pl.pallas_call(
      body,
      grid=(G,),
      in_specs=[
          pl.BlockSpec((2, R, H), lambda i: (0, i, 0)),
          pl.BlockSpec((1, H), lambda i: (0, 0)),
      ],
      out_specs=pl.BlockSpec((R, H), lambda i: (i, 0)),
      out_shape=jax.ShapeDtypeStruct((Npad, H), jnp.float32),
  )(acc, b)


def _tc_final(acc, W3, b3, Npad, G):
  R = Npad // G
  H = acc.shape[2]
  C = W3.shape[1]

  def body(acc_ref, w_ref, b_ref, out_ref):
    agg = acc_ref[0] + acc_ref[1]
    out_ref[...] = jnp.dot(agg, w_ref[...],
                           preferred_element_type=jnp.float32) + b_ref[...]

  return pl.pallas_call(
      body,
      grid=(G,),
      in_specs=[
          pl.BlockSpec((2, R, H), lambda i: (0, i, 0)),
          pl.BlockSpec((H, C), lambda i: (0, 0)),
          pl.BlockSpec((1, C), lambda i: (0, 0)),
      ],
      out_specs=pl.BlockSpec((R, C), lambda i: (i, 0)),
      out_shape=jax.ShapeDtypeStruct((Npad, C), jnp.float32),
  )(acc, W3, b3)


def kernel(x, edge_index, edge_weight, W1, b1, W2, b2, W3, b3):
  N, D = x.shape
  E = edge_index.shape[1]
  H = W1.shape[1]
  C = W3.shape[1]

  Npad = -(-N // 2048) * 2048
  E_tot = E + N
  CH = -(-E_tot // (NW * CE))
  E_pad = NW * CH * CE

  src = edge_index[0]
  dst = edge_index[1]
  loop = jnp.arange(N, dtype=src.dtype)
  pad = E_pad - E_tot
  sE = jnp.concatenate([src, loop, jnp.zeros((pad,), src.dtype)])
  dE = jnp.concatenate([dst, loop, jnp.zeros((pad,), src.dtype)])
  wE = jnp.concatenate([edge_weight, jnp.ones((N,), jnp.float32),
                        jnp.zeros((pad,), jnp.float32)])
  sE3 = sE.reshape(NW, CH, CE)
  dE3 = dE.reshape(NW, CH, CE)
  wE3 = wE.reshape(NW, CH, CE)
  xp = jnp.pad(x, ((0, Npad - N), (0, 0)))

  G = 5

  deg2 = _make_sc_deg(CH, Npad)(dE3, wE3)
  dinv2d, hw1 = _tc1(deg2.reshape(2, Npad // 128, 128), xp, W1, Npad, G)
  dinv = dinv2d.reshape(Npad)
  norm3 = _make_sc_norm(CH)(dinv, sE3, dE3, wE3)

  agg = _make_sc_agg(CH, Npad, H)
  acc1 = agg(hw1, sE3, dE3, norm3)
  hw2 = _tc_layer(acc1, b1.reshape(1, H), W2, Npad, G)
  acc2 = agg(hw2, sE3, dE3, norm3)
  h3 = _tc_relu(acc2, b2.reshape(1, H), Npad, G)
  acc3 = agg(h3, sE3, dE3, norm3)
  out = _tc_final(acc3, W3, b3.reshape(1, C), Npad, G)
  return out[:N]
